# baseline (speedup 1.0000x reference)
import functools

import jax
import jax.numpy as jnp
from jax import lax
from jax.experimental import pallas as pl
from jax.experimental.pallas import tpu as pltpu
from jax.experimental.pallas import tpu_sc as plsc

Q = 65536
D = 128
B = 1024
NW = 32
RPW = Q // NW
CH = 512
NCH = RPW // CH
SH = Q - B

_mesh = plsc.VectorSubcoreMesh(core_axis_name="c", subcore_axis_name="s")


@functools.partial(
    pl.kernel,
    mesh=_mesh,
    out_type=jax.ShapeDtypeStruct((Q, D), jnp.float32),
    scratch_types=[
        pltpu.VMEM((CH, D), jnp.float32),
        pltpu.VMEM((CH, D), jnp.float32),
        pltpu.SemaphoreType.DMA,
        pltpu.SemaphoreType.DMA,
        pltpu.SemaphoreType.DMA,
        pltpu.SemaphoreType.DMA,
    ],
)
def _sc_copy(keys_hbm, old_hbm, b0, b1, sr0, sr1, sw0, sw1):
    wid = lax.axis_index("s") * 2 + lax.axis_index("c")
    base = wid * RPW
    bufs = (b0, b1)
    srs = (sr0, sr1)
    sws = (sw0, sw1)

    reads = {0: pltpu.async_copy(keys_hbm.at[pl.ds(base, CH)], bufs[0], srs[0])}
    writes = {}
    for c in range(NCH):
        bsel = c % 2
        reads[c].wait()
        writes[c] = pltpu.async_copy(
            bufs[bsel], old_hbm.at[pl.ds(base + c * CH, CH)], sws[bsel])
        if c + 1 < NCH:
            nb = (c + 1) % 2
            if c >= 1:
                writes[c - 1].wait()
            reads[c + 1] = pltpu.async_copy(
                keys_hbm.at[pl.ds(base + (c + 1) * CH, CH)], bufs[nb], srs[nb])
    writes[NCH - 2].wait()
    writes[NCH - 1].wait()


def _tc_body(new_ref, keys_ref, out_ref, sem0, sem1):
    shift = pltpu.make_async_copy(
        keys_ref.at[pl.ds(0, SH)], out_ref.at[pl.ds(B, SH)], sem0)
    head = pltpu.make_async_copy(new_ref, out_ref.at[pl.ds(0, B)], sem1)
    shift.start()
    head.start()
    head.wait()
    shift.wait()


_tc_shift = pl.pallas_call(
    _tc_body,
    in_specs=[
        pl.BlockSpec(memory_space=pl.ANY),
        pl.BlockSpec(memory_space=pl.ANY),
    ],
    out_specs=pl.BlockSpec(memory_space=pl.ANY),
    out_shape=jax.ShapeDtypeStruct((Q, D), jnp.float32),
    scratch_shapes=[pltpu.SemaphoreType.DMA, pltpu.SemaphoreType.DMA],
)


def kernel(new_keys, keys):
    old_keys = _sc_copy(keys)
    updated_keys = _tc_shift(new_keys, keys)
    return (old_keys, updated_keys)


# speedup vs baseline: 12.2932x; 12.2932x over previous
import functools

import jax
import jax.numpy as jnp
from jax import lax
from jax.experimental import pallas as pl
from jax.experimental.pallas import tpu as pltpu
from jax.experimental.pallas import tpu_sc as plsc

Q = 65536
D = 128
B = 1024
NW = 32
RPW = Q // NW
CH = 512
NCH = RPW // CH
SH = Q - B

_mesh = plsc.VectorSubcoreMesh(core_axis_name="c", subcore_axis_name="s")


@functools.partial(
    pl.kernel,
    mesh=_mesh,
    out_type=jax.ShapeDtypeStruct((Q, D), jnp.float32),
    scratch_types=[
        pltpu.VMEM((CH, D), jnp.float32),
        pltpu.VMEM((CH, D), jnp.float32),
        pltpu.SemaphoreType.DMA,
        pltpu.SemaphoreType.DMA,
        pltpu.SemaphoreType.DMA,
        pltpu.SemaphoreType.DMA,
    ],
)
def _sc_copy(keys_hbm, old_hbm, b0, b1, sr0, sr1, sw0, sw1):
    wid = lax.axis_index("s") * 2 + lax.axis_index("c")
    base = wid * RPW
    bufs = (b0, b1)
    srs = (sr0, sr1)
    sws = (sw0, sw1)

    reads = {0: pltpu.async_copy(keys_hbm.at[pl.ds(base, CH)], bufs[0], srs[0])}
    writes = {}
    for c in range(NCH):
        bsel = c % 2
        reads[c].wait()
        writes[c] = pltpu.async_copy(
            bufs[bsel], old_hbm.at[pl.ds(base + c * CH, CH)], sws[bsel])
        if c + 1 < NCH:
            nb = (c + 1) % 2
            if c >= 1:
                writes[c - 1].wait()
            reads[c + 1] = pltpu.async_copy(
                keys_hbm.at[pl.ds(base + (c + 1) * CH, CH)], bufs[nb], srs[nb])
    writes[NCH - 2].wait()
    writes[NCH - 1].wait()


CHT = 2016
NCHT = SH // CHT


def _tc_body(new_ref, keys_ref, out_ref, b0, b1, hbuf,
             sr0, sr1, sw0, sw1, sh):
    bufs = (b0, b1)
    srs = (sr0, sr1)
    sws = (sw0, sw1)

    hread = pltpu.make_async_copy(new_ref, hbuf, sh)
    hread.start()

    reads = {0: pltpu.make_async_copy(
        keys_ref.at[pl.ds(0, CHT)], bufs[0], srs[0])}
    reads[0].start()
    writes = {}
    for c in range(NCHT):
        bsel = c % 2
        reads[c].wait()
        writes[c] = pltpu.make_async_copy(
            bufs[bsel], out_ref.at[pl.ds(c * CHT + B, CHT)], sws[bsel])
        writes[c].start()
        if c + 1 < NCHT:
            nb = (c + 1) % 2
            if c >= 1:
                writes[c - 1].wait()
            reads[c + 1] = pltpu.make_async_copy(
                keys_ref.at[pl.ds((c + 1) * CHT, CHT)], bufs[nb], srs[nb])
            reads[c + 1].start()
    hread.wait()
    hwrite = pltpu.make_async_copy(hbuf, out_ref.at[pl.ds(0, B)], sh)
    hwrite.start()
    writes[NCHT - 2].wait()
    writes[NCHT - 1].wait()
    hwrite.wait()


_tc_shift = pl.pallas_call(
    _tc_body,
    in_specs=[
        pl.BlockSpec(memory_space=pl.ANY),
        pl.BlockSpec(memory_space=pl.ANY),
    ],
    out_specs=pl.BlockSpec(memory_space=pl.ANY),
    out_shape=jax.ShapeDtypeStruct((Q, D), jnp.float32),
    scratch_shapes=[
        pltpu.VMEM((CHT, D), jnp.float32),
        pltpu.VMEM((CHT, D), jnp.float32),
        pltpu.VMEM((B, D), jnp.float32),
        pltpu.SemaphoreType.DMA,
        pltpu.SemaphoreType.DMA,
        pltpu.SemaphoreType.DMA,
        pltpu.SemaphoreType.DMA,
        pltpu.SemaphoreType.DMA,
    ],
)


def kernel(new_keys, keys):
    old_keys = _sc_copy(keys)
    updated_keys = _tc_shift(new_keys, keys)
    return (old_keys, updated_keys)


# speedup vs baseline: 17.1410x; 1.3944x over previous
import functools

import jax
import jax.numpy as jnp
from jax import lax
from jax.experimental import pallas as pl
from jax.experimental.pallas import tpu as pltpu
from jax.experimental.pallas import tpu_sc as plsc

Q = 65536
D = 128
B = 1024
NW = 32
RPW = Q // NW
CH = 512
NCH = RPW // CH
SH = Q - B

_mesh = plsc.VectorSubcoreMesh(core_axis_name="c", subcore_axis_name="s")


@functools.partial(
    pl.kernel,
    mesh=_mesh,
    out_type=jax.ShapeDtypeStruct((Q, D), jnp.float32),
    scratch_types=[
        pltpu.VMEM((CH, D), jnp.float32),
        pltpu.VMEM((CH, D), jnp.float32),
        pltpu.SemaphoreType.DMA,
        pltpu.SemaphoreType.DMA,
        pltpu.SemaphoreType.DMA,
        pltpu.SemaphoreType.DMA,
    ],
)
def _sc_copy(keys_hbm, old_hbm, b0, b1, sr0, sr1, sw0, sw1):
    wid = lax.axis_index("s") * 2 + lax.axis_index("c")
    base = wid * RPW
    bufs = (b0, b1)
    srs = (sr0, sr1)
    sws = (sw0, sw1)

    reads = {0: pltpu.async_copy(keys_hbm.at[pl.ds(base, CH)], bufs[0], srs[0])}
    writes = {}
    for c in range(NCH):
        bsel = c % 2
        reads[c].wait()
        writes[c] = pltpu.async_copy(
            bufs[bsel], old_hbm.at[pl.ds(base + c * CH, CH)], sws[bsel])
        if c + 1 < NCH:
            nb = (c + 1) % 2
            if c >= 1:
                writes[c - 1].wait()
            reads[c + 1] = pltpu.async_copy(
                keys_hbm.at[pl.ds(base + (c + 1) * CH, CH)], bufs[nb], srs[nb])
    writes[NCH - 2].wait()
    writes[NCH - 1].wait()


CHT = 2016
NCHT = SH // CHT
NB = 8
RD = 4
WD = 4


def _tc_body(new_ref, keys_ref, out_ref, *rest):
    bufs = rest[:NB]
    hbuf = rest[NB]
    srs = rest[NB + 1:NB + 1 + NB]
    sws = rest[NB + 1 + NB:NB + 1 + 2 * NB]
    sh = rest[NB + 1 + 2 * NB]

    hread = pltpu.make_async_copy(new_ref, hbuf, sh)
    hread.start()

    reads = {}
    writes = {}
    for c in range(min(RD, NCHT)):
        reads[c] = pltpu.make_async_copy(
            keys_ref.at[pl.ds(c * CHT, CHT)], bufs[c % NB], srs[c % NB])
        reads[c].start()
    for c in range(NCHT):
        bsel = c % NB
        reads[c].wait()
        writes[c] = pltpu.make_async_copy(
            bufs[bsel], out_ref.at[pl.ds(c * CHT + B, CHT)], sws[bsel])
        writes[c].start()
        if c - WD >= 0:
            writes[c - WD].wait()
        nxt = c + RD
        if nxt < NCHT:
            reads[nxt] = pltpu.make_async_copy(
                keys_ref.at[pl.ds(nxt * CHT, CHT)], bufs[nxt % NB], srs[nxt % NB])
            reads[nxt].start()
    hread.wait()
    hwrite = pltpu.make_async_copy(hbuf, out_ref.at[pl.ds(0, B)], sh)
    hwrite.start()
    for c in range(max(0, NCHT - WD), NCHT):
        writes[c].wait()
    hwrite.wait()


_tc_shift = pl.pallas_call(
    _tc_body,
    in_specs=[
        pl.BlockSpec(memory_space=pl.ANY),
        pl.BlockSpec(memory_space=pl.ANY),
    ],
    out_specs=pl.BlockSpec(memory_space=pl.ANY),
    out_shape=jax.ShapeDtypeStruct((Q, D), jnp.float32),
    scratch_shapes=(
        [pltpu.VMEM((CHT, D), jnp.float32) for _ in range(NB)]
        + [pltpu.VMEM((B, D), jnp.float32)]
        + [pltpu.SemaphoreType.DMA for _ in range(2 * NB + 1)]
    ),
)


def kernel(new_keys, keys):
    old_keys = _sc_copy(keys)
    updated_keys = _tc_shift(new_keys, keys)
    return (old_keys, updated_keys)
